# baseline (speedup 1.0000x reference)
import functools

import jax
import jax.numpy as jnp
from jax import lax
from jax.experimental import pallas as pl
from jax.experimental.pallas import tpu as pltpu
from jax.experimental.pallas import tpu_sc as plsc

_N = 6400000
_G = 100000
_NC = 2
_NS = 16
_BLK = 4000

_NEG = float("-inf")


def _make_sc_kernel(N, G, NC, NS, BLK, interpret=False):
  NW = NC * NS
  CHUNK = N // NW
  NB = CHUNK // BLK
  VPB = BLK // 16
  GA = G + 96
  WSL = GA // NS
  TRASH = G
  SENT = 2**31 - 1
  WOFF = 16
  LCAP = WOFF + BLK + 128
  NRMAX = (BLK + 127) // 128

  assert CHUNK % BLK == 0 and BLK % 128 == 0 or True
  assert N % NW == 0 and BLK % 16 == 0 and GA % NS == 0 and WSL % 8 == 0

  mesh = plsc.VectorSubcoreMesh(
      core_axis_name="c", subcore_axis_name="s",
      num_cores=NC, num_subcores=NS)

  def body(x_hbm, t_hbm, g_hbm, cnt_out, sum_out, max_out,
           gbuf, tbuf, xbuf, Lg, Lm, Lc, Lp,
           Lg2, Lmi2, Ssum2, Scnt2, Lmv2,
           gsc, msc, csc, psc, tmp16, zbuf, bgv, bmv,
           slots, rgbuf, rmbuf, recg, recm, ridx2, rval2,
           c_sh, s_sh, m_sh, bg_sh, bm_sh, dsem, fsem):
    core = lax.axis_index("c")
    sub = lax.axis_index("s")
    wid = core * NS + sub
    base = pl.multiple_of(wid * CHUNK, 8)
    iota = lax.broadcasted_iota(jnp.int32, (16,), 0)
    trashv = jnp.full((16,), TRASH, jnp.int32)

    def fill(ref, val, n, dtype):
      v = jnp.full((16,), val, dtype)
      def st(i, _):
        ref[pl.ds(i * 16, 16)] = v
        return 0
      lax.fori_loop(0, n // 16, st, 0)

    wsl0 = pl.multiple_of(sub * WSL, 8)
    fill(zbuf, 0.0, WSL, jnp.float32)
    pltpu.sync_copy(zbuf, c_sh.at[pl.ds(wsl0, WSL)])
    pltpu.sync_copy(zbuf, s_sh.at[pl.ds(wsl0, WSL)])
    fill(zbuf, _NEG, WSL, jnp.float32)
    pltpu.sync_copy(zbuf, m_sh.at[pl.ds(wsl0, WSL)])
    gsc[pl.ds(0, 16)] = jnp.full((16,), -1, jnp.int32)
    msc[pl.ds(0, 16)] = jnp.zeros((16,), jnp.float32)
    plsc.subcore_barrier()

    pltpu.sync_copy(
        g_hbm.at[pl.ds(pl.multiple_of(jnp.maximum(base - 16, 0), 8), 16)],
        tmp16)
    prevg = tmp16[pl.ds(0, 16)][15]
    pltpu.sync_copy(g_hbm.at[pl.ds(base, 16)], tmp16)
    firstg = tmp16[pl.ds(0, 16)][0]
    fsg = jnp.where((prevg == firstg) & (wid > 0), firstg, jnp.int32(-1))
    fsgv = jnp.full((16,), fsg, jnp.int32)
    redv = jnp.full((16,), jnp.int32(G + 1) + sub, jnp.int32)

    def issue_loads(bb, pp):
      bel = pl.multiple_of(base + bb * BLK, 8)
      il = bel + BLK == N

      @pl.when(il)
      def _():
        pltpu.async_copy(g_hbm.at[pl.ds(bel, BLK)],
                         gbuf.at[pl.ds(pp * (BLK + 16), BLK)], dsem.at[pp])

      @pl.when(jnp.logical_not(il))
      def _():
        pltpu.async_copy(g_hbm.at[pl.ds(bel, BLK + 16)], gbuf.at[pl.ds(pp * (BLK + 16), BLK + 16)],
                         dsem.at[pp])

      pltpu.async_copy(t_hbm.at[pl.ds(bel, BLK)], tbuf.at[pl.ds(pp * BLK, BLK)], dsem.at[pp])
      pltpu.async_copy(x_hbm.at[pl.ds(bel, BLK)], xbuf.at[pl.ds(pp * BLK, BLK)], dsem.at[pp])

    def drain_loads(bb, pp):
      bel = pl.multiple_of(base + bb * BLK, 8)
      il = bel + BLK == N

      @pl.when(il)
      def _():
        pltpu.make_async_copy(g_hbm.at[pl.ds(bel, BLK)],
                              gbuf.at[pl.ds(pp * (BLK + 16), BLK)],
                              dsem.at[pp]).wait()

      @pl.when(jnp.logical_not(il))
      def _():
        pltpu.make_async_copy(g_hbm.at[pl.ds(bel, BLK + 16)], gbuf.at[pl.ds(pp * (BLK + 16), BLK + 16)],
                              dsem.at[pp]).wait()

      pltpu.make_async_copy(t_hbm.at[pl.ds(bel, BLK)], tbuf.at[pl.ds(pp * BLK, BLK)],
                            dsem.at[pp]).wait()
      pltpu.make_async_copy(x_hbm.at[pl.ds(bel, BLK)], xbuf.at[pl.ds(pp * BLK, BLK)],
                            dsem.at[pp]).wait()

    issue_loads(0, 0)

    def block_body(b, carry):
      pg, pm, cc, pcend, ppos = carry
      p = b & 1
      bel = pl.multiple_of(base + b * BLK, 8)
      is_last = bel + BLK == N

      @pl.when(b + 1 < NB)
      def _():
        issue_loads(b + 1, (b + 1) & 1)

      drain_loads(b, p)

      @pl.when(is_last)
      def _():
        gbuf[pl.ds(p * (BLK + 16) + BLK, 16)] = jnp.full((16,), SENT, jnp.int32)

      Lc[pl.ds(0, 16)] = jnp.full((16,), pcend, jnp.float32)
      Lp[pl.ds(0, 16)] = jnp.full((16,), ppos, jnp.int32)

      def vec_body(v, vc):
        vcc, off = vc
        o = v * 16
        g = gbuf[pl.ds(p * (BLK + 16) + o, 16)]
        gn = gbuf[pl.ds(p * (BLK + 16) + o + 1, 16)]
        t = tbuf[pl.ds(p * BLK + o, 16)]
        x = xbuf[pl.ds(p * BLK + o, 16)]
        m = t
        gsc[pl.ds(16, 16)] = g
        for d in (1, 2, 4, 8):
          msc[pl.ds(16, 16)] = m
          gs = gsc[pl.ds(16 - d, 16)]
          ms = msc[pl.ds(16 - d, 16)]
          m = jnp.where(g == gs, jnp.maximum(m, ms), m)
        c = plsc.cumsum(x) + jnp.full((16,), vcc, jnp.float32)
        e = (g != gn) | (iota == 15)
        offw = WOFF + off
        plsc.store_compressed(Lg.at[pl.ds(offw, 16)], g, mask=e)
        plsc.store_compressed(Lm.at[pl.ds(offw, 16)], m, mask=e)
        plsc.store_compressed(Lc.at[pl.ds(offw, 16)], c, mask=e)
        pos = iota + (b * BLK + o)
        plsc.store_compressed(Lp.at[pl.ds(offw, 16)], pos, mask=e)
        pc = plsc.all_reduce_population_count(e)
        noff = off + pc[0]
        return (c[15], noff)

      cc, off = lax.fori_loop(
          0, VPB, vec_body, (cc, jnp.int32(0)), unroll=4)

      lkv = gbuf[pl.ds(p * (BLK + 16) + BLK - 1, 16)]
      cont = lkv[0] == lkv[1]
      supp = jnp.where(cont, off - 1, jnp.int32(-2))

      for k in range(8):
        Lg[pl.ds(WOFF + off + k * 16, 16)] = trashv

      nrows = lax.shift_right_logical(off + 127, 7)

      def row_body(j, rc):
        rpg, rpm = rc
        for k in range(8):
          p0 = WOFF + j * 128 + k * 16
          lg = Lg[pl.ds(p0, 16)]
          lgn = Lg[pl.ds(p0 + 1, 16)]
          lc = Lc[pl.ds(p0, 16)]
          lcm = Lc[pl.ds(p0 - 1, 16)]
          lp = Lp[pl.ds(p0, 16)]
          lpm = Lp[pl.ds(p0 - 1, 16)]
          lm = Lm[pl.ds(p0, 16)]
          lm = jnp.where(lg == jnp.full((16,), rpg, jnp.int32),
                         jnp.maximum(lm, jnp.full((16,), rpm, jnp.float32)),
                         lm)
          gsc[pl.ds(16, 16)] = lg
          for d in (1, 2, 4, 8):
            msc[pl.ds(16, 16)] = lm
            gs = gsc[pl.ds(16 - d, 16)]
            ms = msc[pl.ds(16 - d, 16)]
            lm = jnp.where(lg == gs, jnp.maximum(lm, ms), lm)
          Lm[pl.ds(p0, 16)] = lm
          ridx = iota + (j * 128 + k * 16)
          is_end = (lg != lgn) & (ridx != supp)
          Lg2[j, pl.ds(k * 16, 16)] = lg
          Lmi2[j, pl.ds(k * 16, 16)] = jnp.where(
              is_end, jnp.where(lg == fsgv, redv, lg), trashv)
          Ssum2[j, pl.ds(k * 16, 16)] = lc - lcm
          Scnt2[j, pl.ds(k * 16, 16)] = (lp - lpm).astype(jnp.float32)
          Lmv2[j, pl.ds(k * 16, 16)] = lm
          rpg = lg[15]
          rpm = lm[15]
        d1 = pltpu.async_copy(Ssum2.at[j], s_sh.at[Lg2.at[j]], fsem,
                              add=True)
        d2 = pltpu.async_copy(Scnt2.at[j], c_sh.at[Lg2.at[j]], fsem,
                              add=True)
        d3 = pltpu.async_copy(Lmv2.at[j], m_sh.at[Lmi2.at[j]], fsem)
        d1.wait()
        d2.wait()
        d3.wait()
        return (rpg, rpm)

      pg, pm = lax.fori_loop(0, nrows, row_body, (pg, pm))

      pcend2 = Lc[pl.ds(off, 16)][15]
      ppos2 = Lp[pl.ds(off, 16)][15]
      pg2 = Lg[pl.ds(off, 16)][15]
      pm2 = Lm[pl.ds(off, 16)][15]
      return (pg2, pm2, cc, pcend2, ppos2)

    cg, cm, cc, pcend, ppos = lax.fori_loop(
        0, NB, block_body,
        (jnp.int32(-1), jnp.float32(_NEG), jnp.float32(0.0),
         jnp.float32(0.0), jnp.int32(-1)))

    del cc, pcend, ppos

    fsg_rec = jnp.where(fsg == -1, TRASH, fsg)
    bg = jnp.where(iota == 0, jnp.full((16,), fsg_rec, jnp.int32),
                   jnp.where(iota == 1, jnp.full((16,), cg, jnp.int32),
                             trashv))
    bm = jnp.where(iota == 1, jnp.full((16,), cm, jnp.float32),
                   jnp.full((16,), _NEG, jnp.float32))
    bgv[pl.ds(0, 16)] = bg
    bmv[pl.ds(0, 16)] = bm
    b8 = pl.multiple_of(sub * 8, 8)
    pltpu.sync_copy(bgv.at[pl.ds(0, 8)], bg_sh.at[pl.ds(b8, 8)])
    pltpu.sync_copy(bmv.at[pl.ds(0, 8)], bm_sh.at[pl.ds(b8, 8)])
    plsc.subcore_barrier()

    @pl.when(sub == 0)
    def _():
      pltpu.sync_copy(bg_sh, rgbuf)
      pltpu.sync_copy(bm_sh, rmbuf)
      pltpu.sync_copy(m_sh.at[pl.ds(G, 32)], slots)
      recg[pl.ds(32, 16)] = jnp.full((16,), SENT, jnp.int32)
      half = lax.shift_right_logical(iota, 1)
      odd = (iota & 1) == 1
      for r in range(2):
        sl = half + r * 8
        src = sl * 8 + (iota & 1)
        gvec = plsc.load_gather(rgbuf, [src])
        recg[pl.ds(16 * r, 16)] = gvec
        m_even = plsc.load_gather(slots, [sl + 1])
        m_odd = plsc.load_gather(rmbuf, [sl * 8 + 1])
        recm[pl.ds(16 * r, 16)] = jnp.where(odd, m_odd, m_even)
      ccg = jnp.int32(-1)
      ccm = _NEG
      for r in range(2):
        g = recg[pl.ds(16 * r, 16)]
        m0 = recm[pl.ds(16 * r, 16)]
        gn = recg[pl.ds(16 * r + 1, 16)]
        m = jnp.where(g == jnp.full((16,), ccg, jnp.int32),
                      jnp.maximum(m0, jnp.full((16,), ccm, jnp.float32)), m0)
        gsc[pl.ds(16, 16)] = g
        for d in (1, 2, 4, 8):
          msc[pl.ds(16, 16)] = m
          gs = gsc[pl.ds(16 - d, 16)]
          ms = msc[pl.ds(16 - d, 16)]
          m = jnp.where(g == gs, jnp.maximum(m, ms), m)
        e = g != gn
        ridx2[r, pl.ds(0, 16)] = jnp.where(e, g, trashv)
        rval2[r, pl.ds(0, 16)] = m
        ccg = g[15]
        ccm = m[15]
      for r in range(2):
        pltpu.sync_copy(rval2.at[r], m_sh.at[ridx2.at[r]])

    plsc.subcore_barrier()

    ob = pl.multiple_of(core * GA + wsl0, 8)
    for sh, out in ((c_sh, cnt_out), (s_sh, sum_out), (m_sh, max_out)):
      pltpu.sync_copy(sh.at[pl.ds(wsl0, WSL)], zbuf)
      pltpu.sync_copy(zbuf, out.at[pl.ds(ob, WSL)])

  f32 = jnp.float32
  i32 = jnp.int32
  out_type = [jax.ShapeDtypeStruct((NC * GA,), f32)] * 3
  scratch = [
      pltpu.VMEM((2 * (BLK + 16),), i32),
      pltpu.VMEM((2 * BLK,), f32),
      pltpu.VMEM((2 * BLK,), f32),
      pltpu.VMEM((LCAP,), i32),
      pltpu.VMEM((LCAP,), f32),
      pltpu.VMEM((LCAP,), f32),
      pltpu.VMEM((LCAP,), i32),
      pltpu.VMEM((NRMAX, 128), i32),
      pltpu.VMEM((NRMAX, 128), i32),
      pltpu.VMEM((NRMAX, 128), f32),
      pltpu.VMEM((NRMAX, 128), f32),
      pltpu.VMEM((NRMAX, 128), f32),
      pltpu.VMEM((32,), i32),
      pltpu.VMEM((32,), f32),
      pltpu.VMEM((16,), f32),
      pltpu.VMEM((16,), i32),
      pltpu.VMEM((16,), i32),
      pltpu.VMEM((WSL,), f32),
      pltpu.VMEM((16,), i32),
      pltpu.VMEM((16,), f32),
      pltpu.VMEM((32,), f32),
      pltpu.VMEM((128,), i32),
      pltpu.VMEM((128,), f32),
      pltpu.VMEM((48,), i32),
      pltpu.VMEM((48,), f32),
      pltpu.VMEM((2, 16), i32),
      pltpu.VMEM((2, 16), f32),
      pltpu.VMEM_SHARED((GA,), f32),
      pltpu.VMEM_SHARED((GA,), f32),
      pltpu.VMEM_SHARED((GA,), f32),
      pltpu.VMEM_SHARED((128,), i32),
      pltpu.VMEM_SHARED((128,), f32),
      pltpu.SemaphoreType.DMA((2,)),
      pltpu.SemaphoreType.DMA,
  ]
  return pl.kernel(body, out_type=out_type, mesh=mesh,
                   scratch_types=scratch,
                   compiler_params=pltpu.CompilerParams(
                       needs_layout_passes=False),
                   interpret=interpret)


def _make_tc_kernel(G, NC, GA, interpret=False):
  BCOL = 5888
  NSTEP = GA // BCOL
  assert NSTEP * BCOL == GA

  def tc_body(c_ref, s_ref, m_ref, out_ref, acc_ref):
    i = pl.program_id(0)

    @pl.when(i == 0)
    def _():
      acc_ref[0] = 0.0
      acc_ref[1] = 0.0

    c = c_ref[0:1, :] + c_ref[1:2, :]
    s = s_ref[0:1, :] + s_ref[1:2, :]
    m = jnp.maximum(m_ref[0:1, :], m_ref[1:2, :])
    col = lax.broadcasted_iota(jnp.int32, (1, BCOL), 1) + i * BCOL
    valid = (col < G) & (c > 0.0)
    mean = jnp.where(valid, s / jnp.maximum(c, 1.0), 0.0)
    tm = jnp.where(valid, m, 0.0)

    def lgs(x):
      return jnp.minimum(x, 0.0) - jnp.log1p(jnp.exp(-jnp.abs(x)))

    per = tm * lgs(mean) + (1.0 - tm) * lgs(1.0 - mean)
    per = jnp.where(valid, per, 0.0)
    acc_ref[0] += jnp.sum(per)
    acc_ref[1] += jnp.sum(valid.astype(jnp.float32))

    @pl.when(i == NSTEP - 1)
    def _():
      out_ref[0, 0] = -acc_ref[0] / acc_ref[1]

  return pl.pallas_call(
      tc_body,
      grid=(NSTEP,),
      in_specs=[pl.BlockSpec((NC, BCOL), lambda i: (0, i))] * 3,
      out_specs=pl.BlockSpec((1, 1), lambda i: (0, 0),
                             memory_space=pltpu.SMEM),
      out_shape=jax.ShapeDtypeStruct((1, 1), jnp.float32),
      scratch_shapes=[pltpu.SMEM((2,), jnp.float32)],
      interpret=interpret,
  )


@functools.lru_cache(maxsize=None)
def _build():
  sc = _make_sc_kernel(_N, _G, _NC, _NS, _BLK)
  tc = _make_tc_kernel(_G, _NC, _G + 96)
  return sc, tc


def kernel(input, target, group_id):
  sc, tc = _build()
  cnt2, sum2, max2 = sc(input.astype(jnp.float32),
                        target.astype(jnp.float32),
                        group_id.astype(jnp.int32))
  ga = _G + 96
  out = tc(cnt2.reshape(_NC, ga), sum2.reshape(_NC, ga),
           max2.reshape(_NC, ga))
  return out[0, 0]


# speedup vs baseline: 1.0007x; 1.0007x over previous
import functools

import jax
import jax.numpy as jnp
from jax import lax
from jax.experimental import pallas as pl
from jax.experimental.pallas import tpu as pltpu
from jax.experimental.pallas import tpu_sc as plsc

_N = 6400000
_G = 100000
_NC = 2
_NS = 16
_BLK = 4000

_NEG = float("-inf")


def _make_sc_kernel(N, G, NC, NS, BLK, interpret=False):
  NW = NC * NS
  CHUNK = N // NW
  NB = CHUNK // BLK
  VPB = BLK // 16
  GA = G + 96
  WSL = GA // NS
  TRASH = G
  SENT = 2**31 - 1
  WOFF = 16
  LCAP = WOFF + BLK + 128
  NRMAX = (BLK + 127) // 128

  assert CHUNK % BLK == 0 and BLK % 128 == 0 or True
  assert N % NW == 0 and BLK % 16 == 0 and GA % NS == 0 and WSL % 8 == 0

  mesh = plsc.VectorSubcoreMesh(
      core_axis_name="c", subcore_axis_name="s",
      num_cores=NC, num_subcores=NS)

  def body(x_hbm, t_hbm, g_hbm, cnt_out, sum_out, max_out,
           gbuf, tbuf, xbuf, Lg, Lm, Lc, Lp,
           Lg2, Lmi2, Ssum2, Scnt2, Lmv2,
           gsc, msc, csc, psc, tmp16, zbuf, bgv, bmv,
           slots, rgbuf, rmbuf, recg, recm, ridx2, rval2,
           c_sh, s_sh, m_sh, bg_sh, bm_sh, dsem, fsem):
    core = lax.axis_index("c")
    sub = lax.axis_index("s")
    wid = core * NS + sub
    base = pl.multiple_of(wid * CHUNK, 8)
    iota = lax.broadcasted_iota(jnp.int32, (16,), 0)
    trashv = jnp.full((16,), TRASH, jnp.int32)

    def fill(ref, val, n, dtype):
      v = jnp.full((16,), val, dtype)
      def st(i, _):
        ref[pl.ds(i * 16, 16)] = v
        return 0
      lax.fori_loop(0, n // 16, st, 0)

    wsl0 = pl.multiple_of(sub * WSL, 8)
    fill(zbuf, 0.0, WSL, jnp.float32)
    pltpu.sync_copy(zbuf, c_sh.at[pl.ds(wsl0, WSL)])
    pltpu.sync_copy(zbuf, s_sh.at[pl.ds(wsl0, WSL)])
    fill(zbuf, _NEG, WSL, jnp.float32)
    pltpu.sync_copy(zbuf, m_sh.at[pl.ds(wsl0, WSL)])
    for u in range(5):
      gsc[pl.ds(u * 32, 16)] = jnp.full((16,), -1, jnp.int32)
      msc[pl.ds(u * 32, 16)] = jnp.zeros((16,), jnp.float32)
    plsc.subcore_barrier()

    pltpu.sync_copy(
        g_hbm.at[pl.ds(pl.multiple_of(jnp.maximum(base - 16, 0), 8), 16)],
        tmp16)
    prevg = tmp16[pl.ds(0, 16)][15]
    pltpu.sync_copy(g_hbm.at[pl.ds(base, 16)], tmp16)
    firstg = tmp16[pl.ds(0, 16)][0]
    fsg = jnp.where((prevg == firstg) & (wid > 0), firstg, jnp.int32(-1))
    fsgv = jnp.full((16,), fsg, jnp.int32)
    redv = jnp.full((16,), jnp.int32(G + 1) + sub, jnp.int32)

    def issue_loads(bb, pp):
      bel = pl.multiple_of(base + bb * BLK, 8)
      il = bel + BLK == N

      @pl.when(il)
      def _():
        pltpu.async_copy(g_hbm.at[pl.ds(bel, BLK)],
                         gbuf.at[pl.ds(pp * (BLK + 16), BLK)], dsem.at[pp])

      @pl.when(jnp.logical_not(il))
      def _():
        pltpu.async_copy(g_hbm.at[pl.ds(bel, BLK + 16)], gbuf.at[pl.ds(pp * (BLK + 16), BLK + 16)],
                         dsem.at[pp])

      pltpu.async_copy(t_hbm.at[pl.ds(bel, BLK)], tbuf.at[pl.ds(pp * BLK, BLK)], dsem.at[pp])
      pltpu.async_copy(x_hbm.at[pl.ds(bel, BLK)], xbuf.at[pl.ds(pp * BLK, BLK)], dsem.at[pp])

    def drain_loads(bb, pp):
      bel = pl.multiple_of(base + bb * BLK, 8)
      il = bel + BLK == N

      @pl.when(il)
      def _():
        pltpu.make_async_copy(g_hbm.at[pl.ds(bel, BLK)],
                              gbuf.at[pl.ds(pp * (BLK + 16), BLK)],
                              dsem.at[pp]).wait()

      @pl.when(jnp.logical_not(il))
      def _():
        pltpu.make_async_copy(g_hbm.at[pl.ds(bel, BLK + 16)], gbuf.at[pl.ds(pp * (BLK + 16), BLK + 16)],
                              dsem.at[pp]).wait()

      pltpu.make_async_copy(t_hbm.at[pl.ds(bel, BLK)], tbuf.at[pl.ds(pp * BLK, BLK)],
                            dsem.at[pp]).wait()
      pltpu.make_async_copy(x_hbm.at[pl.ds(bel, BLK)], xbuf.at[pl.ds(pp * BLK, BLK)],
                            dsem.at[pp]).wait()

    issue_loads(0, 0)

    def block_body(b, carry):
      pg, pm, cc, pcend, ppos = carry
      p = b & 1
      bel = pl.multiple_of(base + b * BLK, 8)
      is_last = bel + BLK == N

      @pl.when(b + 1 < NB)
      def _():
        issue_loads(b + 1, (b + 1) & 1)

      drain_loads(b, p)

      @pl.when(is_last)
      def _():
        gbuf[pl.ds(p * (BLK + 16) + BLK, 16)] = jnp.full((16,), SENT, jnp.int32)

      Lc[pl.ds(0, 16)] = jnp.full((16,), pcend, jnp.float32)
      Lp[pl.ds(0, 16)] = jnp.full((16,), ppos, jnp.int32)

      def vec_body(v, vc):
        vcc, off = vc
        for u in range(5):
          o = (v * 5 + u) * 16
          su = u * 32
          g = gbuf[pl.ds(p * (BLK + 16) + o, 16)]
          gn = gbuf[pl.ds(p * (BLK + 16) + o + 1, 16)]
          t = tbuf[pl.ds(p * BLK + o, 16)]
          x = xbuf[pl.ds(p * BLK + o, 16)]
          m = t
          gsc[pl.ds(su + 16, 16)] = g
          for d in (1, 2, 4, 8):
            msc[pl.ds(su + 16, 16)] = m
            gs = gsc[pl.ds(su + 16 - d, 16)]
            ms = msc[pl.ds(su + 16 - d, 16)]
            m = jnp.where(g == gs, jnp.maximum(m, ms), m)
          c = plsc.cumsum(x) + jnp.full((16,), vcc, jnp.float32)
          e = (g != gn) | (iota == 15)
          offw = WOFF + off
          plsc.store_compressed(Lg.at[pl.ds(offw, 16)], g, mask=e)
          plsc.store_compressed(Lm.at[pl.ds(offw, 16)], m, mask=e)
          plsc.store_compressed(Lc.at[pl.ds(offw, 16)], c, mask=e)
          pos = iota + (b * BLK + o)
          plsc.store_compressed(Lp.at[pl.ds(offw, 16)], pos, mask=e)
          pc = plsc.all_reduce_population_count(e)
          off = off + pc[0]
          vcc = c[15]
        return (vcc, off)

      cc, off = lax.fori_loop(
          0, VPB // 5, vec_body, (cc, jnp.int32(0)))

      lkv = gbuf[pl.ds(p * (BLK + 16) + BLK - 1, 16)]
      cont = lkv[0] == lkv[1]
      supp = jnp.where(cont, off - 1, jnp.int32(-2))

      for k in range(8):
        Lg[pl.ds(WOFF + off + k * 16, 16)] = trashv

      nrows = lax.shift_right_logical(off + 127, 7)

      def row_body(j, rc):
        rpg, rpm = rc
        for k in range(8):
          p0 = WOFF + j * 128 + k * 16
          lg = Lg[pl.ds(p0, 16)]
          lgn = Lg[pl.ds(p0 + 1, 16)]
          lc = Lc[pl.ds(p0, 16)]
          lcm = Lc[pl.ds(p0 - 1, 16)]
          lp = Lp[pl.ds(p0, 16)]
          lpm = Lp[pl.ds(p0 - 1, 16)]
          lm = Lm[pl.ds(p0, 16)]
          lm = jnp.where(lg == jnp.full((16,), rpg, jnp.int32),
                         jnp.maximum(lm, jnp.full((16,), rpm, jnp.float32)),
                         lm)
          gsc[pl.ds(16, 16)] = lg
          for d in (1, 2, 4, 8):
            msc[pl.ds(16, 16)] = lm
            gs = gsc[pl.ds(16 - d, 16)]
            ms = msc[pl.ds(16 - d, 16)]
            lm = jnp.where(lg == gs, jnp.maximum(lm, ms), lm)
          Lm[pl.ds(p0, 16)] = lm
          ridx = iota + (j * 128 + k * 16)
          is_end = (lg != lgn) & (ridx != supp)
          Lg2[j, pl.ds(k * 16, 16)] = lg
          Lmi2[j, pl.ds(k * 16, 16)] = jnp.where(
              is_end, jnp.where(lg == fsgv, redv, lg), trashv)
          Ssum2[j, pl.ds(k * 16, 16)] = lc - lcm
          Scnt2[j, pl.ds(k * 16, 16)] = (lp - lpm).astype(jnp.float32)
          Lmv2[j, pl.ds(k * 16, 16)] = lm
          rpg = lg[15]
          rpm = lm[15]
        d1 = pltpu.async_copy(Ssum2.at[j], s_sh.at[Lg2.at[j]], fsem,
                              add=True)
        d2 = pltpu.async_copy(Scnt2.at[j], c_sh.at[Lg2.at[j]], fsem,
                              add=True)
        d3 = pltpu.async_copy(Lmv2.at[j], m_sh.at[Lmi2.at[j]], fsem)
        d1.wait()
        d2.wait()
        d3.wait()
        return (rpg, rpm)

      pg, pm = lax.fori_loop(0, nrows, row_body, (pg, pm))

      pcend2 = Lc[pl.ds(off, 16)][15]
      ppos2 = Lp[pl.ds(off, 16)][15]
      pg2 = Lg[pl.ds(off, 16)][15]
      pm2 = Lm[pl.ds(off, 16)][15]
      return (pg2, pm2, cc, pcend2, ppos2)

    cg, cm, cc, pcend, ppos = lax.fori_loop(
        0, NB, block_body,
        (jnp.int32(-1), jnp.float32(_NEG), jnp.float32(0.0),
         jnp.float32(0.0), jnp.int32(-1)))

    del cc, pcend, ppos

    fsg_rec = jnp.where(fsg == -1, TRASH, fsg)
    bg = jnp.where(iota == 0, jnp.full((16,), fsg_rec, jnp.int32),
                   jnp.where(iota == 1, jnp.full((16,), cg, jnp.int32),
                             trashv))
    bm = jnp.where(iota == 1, jnp.full((16,), cm, jnp.float32),
                   jnp.full((16,), _NEG, jnp.float32))
    bgv[pl.ds(0, 16)] = bg
    bmv[pl.ds(0, 16)] = bm
    b8 = pl.multiple_of(sub * 8, 8)
    pltpu.sync_copy(bgv.at[pl.ds(0, 8)], bg_sh.at[pl.ds(b8, 8)])
    pltpu.sync_copy(bmv.at[pl.ds(0, 8)], bm_sh.at[pl.ds(b8, 8)])
    plsc.subcore_barrier()

    @pl.when(sub == 0)
    def _():
      pltpu.sync_copy(bg_sh, rgbuf)
      pltpu.sync_copy(bm_sh, rmbuf)
      pltpu.sync_copy(m_sh.at[pl.ds(G, 32)], slots)
      recg[pl.ds(32, 16)] = jnp.full((16,), SENT, jnp.int32)
      half = lax.shift_right_logical(iota, 1)
      odd = (iota & 1) == 1
      for r in range(2):
        sl = half + r * 8
        src = sl * 8 + (iota & 1)
        gvec = plsc.load_gather(rgbuf, [src])
        recg[pl.ds(16 * r, 16)] = gvec
        m_even = plsc.load_gather(slots, [sl + 1])
        m_odd = plsc.load_gather(rmbuf, [sl * 8 + 1])
        recm[pl.ds(16 * r, 16)] = jnp.where(odd, m_odd, m_even)
      ccg = jnp.int32(-1)
      ccm = _NEG
      for r in range(2):
        g = recg[pl.ds(16 * r, 16)]
        m0 = recm[pl.ds(16 * r, 16)]
        gn = recg[pl.ds(16 * r + 1, 16)]
        m = jnp.where(g == jnp.full((16,), ccg, jnp.int32),
                      jnp.maximum(m0, jnp.full((16,), ccm, jnp.float32)), m0)
        gsc[pl.ds(16, 16)] = g
        for d in (1, 2, 4, 8):
          msc[pl.ds(16, 16)] = m
          gs = gsc[pl.ds(16 - d, 16)]
          ms = msc[pl.ds(16 - d, 16)]
          m = jnp.where(g == gs, jnp.maximum(m, ms), m)
        e = g != gn
        ridx2[r, pl.ds(0, 16)] = jnp.where(e, g, trashv)
        rval2[r, pl.ds(0, 16)] = m
        ccg = g[15]
        ccm = m[15]
      for r in range(2):
        pltpu.sync_copy(rval2.at[r], m_sh.at[ridx2.at[r]])

    plsc.subcore_barrier()

    ob = pl.multiple_of(core * GA + wsl0, 8)
    for sh, out in ((c_sh, cnt_out), (s_sh, sum_out), (m_sh, max_out)):
      pltpu.sync_copy(sh.at[pl.ds(wsl0, WSL)], zbuf)
      pltpu.sync_copy(zbuf, out.at[pl.ds(ob, WSL)])

  f32 = jnp.float32
  i32 = jnp.int32
  out_type = [jax.ShapeDtypeStruct((NC * GA,), f32)] * 3
  scratch = [
      pltpu.VMEM((2 * (BLK + 16),), i32),
      pltpu.VMEM((2 * BLK,), f32),
      pltpu.VMEM((2 * BLK,), f32),
      pltpu.VMEM((LCAP,), i32),
      pltpu.VMEM((LCAP,), f32),
      pltpu.VMEM((LCAP,), f32),
      pltpu.VMEM((LCAP,), i32),
      pltpu.VMEM((NRMAX, 128), i32),
      pltpu.VMEM((NRMAX, 128), i32),
      pltpu.VMEM((NRMAX, 128), f32),
      pltpu.VMEM((NRMAX, 128), f32),
      pltpu.VMEM((NRMAX, 128), f32),
      pltpu.VMEM((160,), i32),
      pltpu.VMEM((160,), f32),
      pltpu.VMEM((16,), f32),
      pltpu.VMEM((16,), i32),
      pltpu.VMEM((16,), i32),
      pltpu.VMEM((WSL,), f32),
      pltpu.VMEM((16,), i32),
      pltpu.VMEM((16,), f32),
      pltpu.VMEM((32,), f32),
      pltpu.VMEM((128,), i32),
      pltpu.VMEM((128,), f32),
      pltpu.VMEM((48,), i32),
      pltpu.VMEM((48,), f32),
      pltpu.VMEM((2, 16), i32),
      pltpu.VMEM((2, 16), f32),
      pltpu.VMEM_SHARED((GA,), f32),
      pltpu.VMEM_SHARED((GA,), f32),
      pltpu.VMEM_SHARED((GA,), f32),
      pltpu.VMEM_SHARED((128,), i32),
      pltpu.VMEM_SHARED((128,), f32),
      pltpu.SemaphoreType.DMA((2,)),
      pltpu.SemaphoreType.DMA,
  ]
  return pl.kernel(body, out_type=out_type, mesh=mesh,
                   scratch_types=scratch,
                   compiler_params=pltpu.CompilerParams(
                       needs_layout_passes=False),
                   interpret=interpret)


def _make_tc_kernel(G, NC, GA, interpret=False):
  BCOL = 5888
  NSTEP = GA // BCOL
  assert NSTEP * BCOL == GA

  def tc_body(c_ref, s_ref, m_ref, out_ref, acc_ref):
    i = pl.program_id(0)

    @pl.when(i == 0)
    def _():
      acc_ref[0] = 0.0
      acc_ref[1] = 0.0

    c = c_ref[0:1, :] + c_ref[1:2, :]
    s = s_ref[0:1, :] + s_ref[1:2, :]
    m = jnp.maximum(m_ref[0:1, :], m_ref[1:2, :])
    col = lax.broadcasted_iota(jnp.int32, (1, BCOL), 1) + i * BCOL
    valid = (col < G) & (c > 0.0)
    mean = jnp.where(valid, s / jnp.maximum(c, 1.0), 0.0)
    tm = jnp.where(valid, m, 0.0)

    def lgs(x):
      return jnp.minimum(x, 0.0) - jnp.log1p(jnp.exp(-jnp.abs(x)))

    per = tm * lgs(mean) + (1.0 - tm) * lgs(1.0 - mean)
    per = jnp.where(valid, per, 0.0)
    acc_ref[0] += jnp.sum(per)
    acc_ref[1] += jnp.sum(valid.astype(jnp.float32))

    @pl.when(i == NSTEP - 1)
    def _():
      out_ref[0, 0] = -acc_ref[0] / acc_ref[1]

  return pl.pallas_call(
      tc_body,
      grid=(NSTEP,),
      in_specs=[pl.BlockSpec((NC, BCOL), lambda i: (0, i))] * 3,
      out_specs=pl.BlockSpec((1, 1), lambda i: (0, 0),
                             memory_space=pltpu.SMEM),
      out_shape=jax.ShapeDtypeStruct((1, 1), jnp.float32),
      scratch_shapes=[pltpu.SMEM((2,), jnp.float32)],
      interpret=interpret,
  )


@functools.lru_cache(maxsize=None)
def _build():
  sc = _make_sc_kernel(_N, _G, _NC, _NS, _BLK)
  tc = _make_tc_kernel(_G, _NC, _G + 96)
  return sc, tc


def kernel(input, target, group_id):
  sc, tc = _build()
  cnt2, sum2, max2 = sc(input.astype(jnp.float32),
                        target.astype(jnp.float32),
                        group_id.astype(jnp.int32))
  ga = _G + 96
  out = tc(cnt2.reshape(_NC, ga), sum2.reshape(_NC, ga),
           max2.reshape(_NC, ga))
  return out[0, 0]


# speedup vs baseline: 1.4298x; 1.4287x over previous
import functools

import jax
import jax.numpy as jnp
from jax import lax
from jax.experimental import pallas as pl
from jax.experimental.pallas import tpu as pltpu
from jax.experimental.pallas import tpu_sc as plsc

_N = 6400000
_G = 100000
_NC = 2
_NS = 16
_BLK = 4000

_NEG = float("-inf")


def _make_sc_kernel(N, G, NC, NS, BLK, interpret=False):
  NW = NC * NS
  CHUNK = N // NW
  NB = CHUNK // BLK
  VPB = BLK // 16
  GA = G + 96
  WSL = GA // NS
  TRASH = G
  SENT = 2**31 - 1
  WOFF = 16
  LCAP = WOFF + BLK + 128
  NRMAX = (BLK + 127) // 128

  assert CHUNK % BLK == 0 and BLK % 128 == 0 or True
  assert N % NW == 0 and BLK % 16 == 0 and GA % NS == 0 and WSL % 8 == 0

  mesh = plsc.VectorSubcoreMesh(
      core_axis_name="c", subcore_axis_name="s",
      num_cores=NC, num_subcores=NS)

  def body(x_hbm, t_hbm, g_hbm, cnt_out, sum_out, max_out,
           gbuf, tbuf, xbuf, Lg, Lm, Lc, Lp,
           Lg2, Lmi2, Ssum2, Scnt2, Lmv2,
           gsc, msc, csc, psc, tmp16, zbuf, bgv, bmv,
           slots, rgbuf, rmbuf, recg, recm, ridx2, rval2,
           c_sh, s_sh, m_sh, bg_sh, bm_sh, dsem, fsem):
    core = lax.axis_index("c")
    sub = lax.axis_index("s")
    wid = core * NS + sub
    base = pl.multiple_of(wid * CHUNK, 8)
    iota = lax.broadcasted_iota(jnp.int32, (16,), 0)
    trashv = jnp.full((16,), TRASH, jnp.int32)

    def fill(ref, val, n, dtype):
      v = jnp.full((16,), val, dtype)
      def st(i, _):
        ref[pl.ds(i * 16, 16)] = v
        return 0
      lax.fori_loop(0, n // 16, st, 0)

    wsl0 = pl.multiple_of(sub * WSL, 8)
    fill(zbuf, 0.0, WSL, jnp.float32)
    pltpu.sync_copy(zbuf, c_sh.at[pl.ds(wsl0, WSL)])
    pltpu.sync_copy(zbuf, s_sh.at[pl.ds(wsl0, WSL)])
    fill(zbuf, _NEG, WSL, jnp.float32)
    pltpu.sync_copy(zbuf, m_sh.at[pl.ds(wsl0, WSL)])
    for u in range(5):
      gsc[pl.ds(u * 32, 16)] = jnp.full((16,), -1, jnp.int32)
      msc[pl.ds(u * 32, 16)] = jnp.zeros((16,), jnp.float32)
    plsc.subcore_barrier()

    pltpu.sync_copy(
        g_hbm.at[pl.ds(pl.multiple_of(jnp.maximum(base - 16, 0), 8), 16)],
        tmp16)
    prevg = tmp16[pl.ds(0, 16)][15]
    pltpu.sync_copy(g_hbm.at[pl.ds(base, 16)], tmp16)
    firstg = tmp16[pl.ds(0, 16)][0]
    fsg = jnp.where((prevg == firstg) & (wid > 0), firstg, jnp.int32(-1))
    fsgv = jnp.full((16,), fsg, jnp.int32)
    redv = jnp.full((16,), jnp.int32(G + 1) + sub, jnp.int32)

    def issue_loads(bb, pp):
      bel = pl.multiple_of(base + bb * BLK, 8)
      il = bel + BLK == N

      @pl.when(il)
      def _():
        pltpu.async_copy(g_hbm.at[pl.ds(bel, BLK)],
                         gbuf.at[pl.ds(pp * (BLK + 16), BLK)], dsem.at[pp])

      @pl.when(jnp.logical_not(il))
      def _():
        pltpu.async_copy(g_hbm.at[pl.ds(bel, BLK + 16)], gbuf.at[pl.ds(pp * (BLK + 16), BLK + 16)],
                         dsem.at[pp])

      pltpu.async_copy(t_hbm.at[pl.ds(bel, BLK)], tbuf.at[pl.ds(pp * BLK, BLK)], dsem.at[pp])
      pltpu.async_copy(x_hbm.at[pl.ds(bel, BLK)], xbuf.at[pl.ds(pp * BLK, BLK)], dsem.at[pp])

    def drain_loads(bb, pp):
      bel = pl.multiple_of(base + bb * BLK, 8)
      il = bel + BLK == N

      @pl.when(il)
      def _():
        pltpu.make_async_copy(g_hbm.at[pl.ds(bel, BLK)],
                              gbuf.at[pl.ds(pp * (BLK + 16), BLK)],
                              dsem.at[pp]).wait()

      @pl.when(jnp.logical_not(il))
      def _():
        pltpu.make_async_copy(g_hbm.at[pl.ds(bel, BLK + 16)], gbuf.at[pl.ds(pp * (BLK + 16), BLK + 16)],
                              dsem.at[pp]).wait()

      pltpu.make_async_copy(t_hbm.at[pl.ds(bel, BLK)], tbuf.at[pl.ds(pp * BLK, BLK)],
                            dsem.at[pp]).wait()
      pltpu.make_async_copy(x_hbm.at[pl.ds(bel, BLK)], xbuf.at[pl.ds(pp * BLK, BLK)],
                            dsem.at[pp]).wait()

    issue_loads(0, 0)

    def block_body(b, carry):
      pg, pm, cc, pcend, ppos = carry
      p = b & 1
      bel = pl.multiple_of(base + b * BLK, 8)
      is_last = bel + BLK == N

      @pl.when(b + 1 < NB)
      def _():
        issue_loads(b + 1, (b + 1) & 1)

      drain_loads(b, p)

      @pl.when(is_last)
      def _():
        gbuf[pl.ds(p * (BLK + 16) + BLK, 16)] = jnp.full((16,), SENT, jnp.int32)

      Lc[pl.ds(0, 16)] = jnp.full((16,), pcend, jnp.float32)
      Lp[pl.ds(0, 16)] = jnp.full((16,), ppos, jnp.int32)

      def vec_body(v, vc):
        vcc, off = vc
        for u in range(5):
          o = (v * 5 + u) * 16
          su = u * 32
          g = gbuf[pl.ds(p * (BLK + 16) + o, 16)]
          gn = gbuf[pl.ds(p * (BLK + 16) + o + 1, 16)]
          t = tbuf[pl.ds(p * BLK + o, 16)]
          x = xbuf[pl.ds(p * BLK + o, 16)]
          enc = lax.shift_left(g, 14) + (t * 16384.0).astype(jnp.int32)
          m = plsc.cummax(enc)
          c = plsc.cumsum(x) + jnp.full((16,), vcc, jnp.float32)
          e = (g != gn) | (iota == 15)
          offw = WOFF + off
          plsc.store_compressed(Lg.at[pl.ds(offw, 16)], g, mask=e)
          plsc.store_compressed(Lm.at[pl.ds(offw, 16)], m, mask=e)
          plsc.store_compressed(Lc.at[pl.ds(offw, 16)], c, mask=e)
          pos = iota + (b * BLK + o)
          plsc.store_compressed(Lp.at[pl.ds(offw, 16)], pos, mask=e)
          pc = plsc.all_reduce_population_count(e)
          off = off + pc[0]
          vcc = c[15]
        return (vcc, off)

      cc, off = lax.fori_loop(
          0, VPB // 5, vec_body, (cc, jnp.int32(0)))

      lkv = gbuf[pl.ds(p * (BLK + 16) + BLK - 1, 16)]
      cont = lkv[0] == lkv[1]
      supp = jnp.where(cont, off - 1, jnp.int32(-2))

      for k in range(8):
        Lg[pl.ds(WOFF + off + k * 16, 16)] = trashv

      nrows = lax.shift_right_logical(off + 127, 7)

      def row_body(j, rpm):
        for k in range(8):
          p0 = WOFF + j * 128 + k * 16
          lg = Lg[pl.ds(p0, 16)]
          lgn = Lg[pl.ds(p0 + 1, 16)]
          lc = Lc[pl.ds(p0, 16)]
          lcm = Lc[pl.ds(p0 - 1, 16)]
          lp = Lp[pl.ds(p0, 16)]
          lpm = Lp[pl.ds(p0 - 1, 16)]
          lm = Lm[pl.ds(p0, 16)]
          lm = jnp.maximum(lm, jnp.full((16,), rpm, jnp.int32))
          lm = plsc.cummax(lm)
          Lm[pl.ds(p0, 16)] = lm
          ridx = iota + (j * 128 + k * 16)
          is_end = (lg != lgn) & (ridx != supp)
          Lg2[j, pl.ds(k * 16, 16)] = lg
          Lmi2[j, pl.ds(k * 16, 16)] = jnp.where(
              is_end, jnp.where(lg == fsgv, redv, lg), trashv)
          Ssum2[j, pl.ds(k * 16, 16)] = lc - lcm
          Scnt2[j, pl.ds(k * 16, 16)] = (lp - lpm).astype(jnp.float32)
          Lmv2[j, pl.ds(k * 16, 16)] = (
              (lm & 16383).astype(jnp.float32) * (1.0 / 16384.0))
          rpm = lm[15]
        d1 = pltpu.async_copy(Ssum2.at[j], s_sh.at[Lg2.at[j]], fsem,
                              add=True)
        d2 = pltpu.async_copy(Scnt2.at[j], c_sh.at[Lg2.at[j]], fsem,
                              add=True)
        d3 = pltpu.async_copy(Lmv2.at[j], m_sh.at[Lmi2.at[j]], fsem)
        d1.wait()
        d2.wait()
        d3.wait()
        return rpm

      lax.fori_loop(0, nrows, row_body, pm)

      pcend2 = Lc[pl.ds(off, 16)][15]
      ppos2 = Lp[pl.ds(off, 16)][15]
      pg2 = Lg[pl.ds(off, 16)][15]
      pm2 = Lm[pl.ds(off, 16)][15]
      return (pg2, pm2, cc, pcend2, ppos2)

    cg, cme, cc, pcend, ppos = lax.fori_loop(
        0, NB, block_body,
        (jnp.int32(-1), jnp.int32(-1), jnp.float32(0.0),
         jnp.float32(0.0), jnp.int32(-1)))
    cm = (cme & 16383).astype(jnp.float32) * (1.0 / 16384.0)

    del cc, pcend, ppos

    fsg_rec = jnp.where(fsg == -1, TRASH, fsg)
    bg = jnp.where(iota == 0, jnp.full((16,), fsg_rec, jnp.int32),
                   jnp.where(iota == 1, jnp.full((16,), cg, jnp.int32),
                             trashv))
    bm = jnp.where(iota == 1, jnp.full((16,), cm, jnp.float32),
                   jnp.full((16,), _NEG, jnp.float32))
    bgv[pl.ds(0, 16)] = bg
    bmv[pl.ds(0, 16)] = bm
    b8 = pl.multiple_of(sub * 8, 8)
    pltpu.sync_copy(bgv.at[pl.ds(0, 8)], bg_sh.at[pl.ds(b8, 8)])
    pltpu.sync_copy(bmv.at[pl.ds(0, 8)], bm_sh.at[pl.ds(b8, 8)])
    plsc.subcore_barrier()

    @pl.when(sub == 0)
    def _():
      pltpu.sync_copy(bg_sh, rgbuf)
      pltpu.sync_copy(bm_sh, rmbuf)
      pltpu.sync_copy(m_sh.at[pl.ds(G, 32)], slots)
      recg[pl.ds(32, 16)] = jnp.full((16,), SENT, jnp.int32)
      half = lax.shift_right_logical(iota, 1)
      odd = (iota & 1) == 1
      for r in range(2):
        sl = half + r * 8
        src = sl * 8 + (iota & 1)
        gvec = plsc.load_gather(rgbuf, [src])
        recg[pl.ds(16 * r, 16)] = gvec
        m_even = plsc.load_gather(slots, [sl + 1])
        m_odd = plsc.load_gather(rmbuf, [sl * 8 + 1])
        recm[pl.ds(16 * r, 16)] = jnp.where(odd, m_odd, m_even)
      ccg = jnp.int32(-1)
      ccm = _NEG
      for r in range(2):
        g = recg[pl.ds(16 * r, 16)]
        m0 = recm[pl.ds(16 * r, 16)]
        gn = recg[pl.ds(16 * r + 1, 16)]
        m = jnp.where(g == jnp.full((16,), ccg, jnp.int32),
                      jnp.maximum(m0, jnp.full((16,), ccm, jnp.float32)), m0)
        gsc[pl.ds(16, 16)] = g
        for d in (1, 2, 4, 8):
          msc[pl.ds(16, 16)] = m
          gs = gsc[pl.ds(16 - d, 16)]
          ms = msc[pl.ds(16 - d, 16)]
          m = jnp.where(g == gs, jnp.maximum(m, ms), m)
        e = g != gn
        ridx2[r, pl.ds(0, 16)] = jnp.where(e, g, trashv)
        rval2[r, pl.ds(0, 16)] = m
        ccg = g[15]
        ccm = m[15]
      for r in range(2):
        pltpu.sync_copy(rval2.at[r], m_sh.at[ridx2.at[r]])

    plsc.subcore_barrier()

    ob = pl.multiple_of(core * GA + wsl0, 8)
    for sh, out in ((c_sh, cnt_out), (s_sh, sum_out), (m_sh, max_out)):
      pltpu.sync_copy(sh.at[pl.ds(wsl0, WSL)], zbuf)
      pltpu.sync_copy(zbuf, out.at[pl.ds(ob, WSL)])

  f32 = jnp.float32
  i32 = jnp.int32
  out_type = [jax.ShapeDtypeStruct((NC * GA,), f32)] * 3
  scratch = [
      pltpu.VMEM((2 * (BLK + 16),), i32),
      pltpu.VMEM((2 * BLK,), f32),
      pltpu.VMEM((2 * BLK,), f32),
      pltpu.VMEM((LCAP,), i32),
      pltpu.VMEM((LCAP,), i32),
      pltpu.VMEM((LCAP,), f32),
      pltpu.VMEM((LCAP,), i32),
      pltpu.VMEM((NRMAX, 128), i32),
      pltpu.VMEM((NRMAX, 128), i32),
      pltpu.VMEM((NRMAX, 128), f32),
      pltpu.VMEM((NRMAX, 128), f32),
      pltpu.VMEM((NRMAX, 128), f32),
      pltpu.VMEM((160,), i32),
      pltpu.VMEM((160,), f32),
      pltpu.VMEM((16,), f32),
      pltpu.VMEM((16,), i32),
      pltpu.VMEM((16,), i32),
      pltpu.VMEM((WSL,), f32),
      pltpu.VMEM((16,), i32),
      pltpu.VMEM((16,), f32),
      pltpu.VMEM((32,), f32),
      pltpu.VMEM((128,), i32),
      pltpu.VMEM((128,), f32),
      pltpu.VMEM((48,), i32),
      pltpu.VMEM((48,), f32),
      pltpu.VMEM((2, 16), i32),
      pltpu.VMEM((2, 16), f32),
      pltpu.VMEM_SHARED((GA,), f32),
      pltpu.VMEM_SHARED((GA,), f32),
      pltpu.VMEM_SHARED((GA,), f32),
      pltpu.VMEM_SHARED((128,), i32),
      pltpu.VMEM_SHARED((128,), f32),
      pltpu.SemaphoreType.DMA((2,)),
      pltpu.SemaphoreType.DMA,
  ]
  return pl.kernel(body, out_type=out_type, mesh=mesh,
                   scratch_types=scratch,
                   compiler_params=pltpu.CompilerParams(
                       needs_layout_passes=False),
                   interpret=interpret)


def _make_tc_kernel(G, NC, GA, interpret=False):
  BCOL = 5888
  NSTEP = GA // BCOL
  assert NSTEP * BCOL == GA

  def tc_body(c_ref, s_ref, m_ref, out_ref, acc_ref):
    i = pl.program_id(0)

    @pl.when(i == 0)
    def _():
      acc_ref[0] = 0.0
      acc_ref[1] = 0.0

    c = c_ref[0:1, :] + c_ref[1:2, :]
    s = s_ref[0:1, :] + s_ref[1:2, :]
    m = jnp.maximum(m_ref[0:1, :], m_ref[1:2, :])
    col = lax.broadcasted_iota(jnp.int32, (1, BCOL), 1) + i * BCOL
    valid = (col < G) & (c > 0.0)
    mean = jnp.where(valid, s / jnp.maximum(c, 1.0), 0.0)
    tm = jnp.where(valid, m, 0.0)

    def lgs(x):
      return jnp.minimum(x, 0.0) - jnp.log1p(jnp.exp(-jnp.abs(x)))

    per = tm * lgs(mean) + (1.0 - tm) * lgs(1.0 - mean)
    per = jnp.where(valid, per, 0.0)
    acc_ref[0] += jnp.sum(per)
    acc_ref[1] += jnp.sum(valid.astype(jnp.float32))

    @pl.when(i == NSTEP - 1)
    def _():
      out_ref[0, 0] = -acc_ref[0] / acc_ref[1]

  return pl.pallas_call(
      tc_body,
      grid=(NSTEP,),
      in_specs=[pl.BlockSpec((NC, BCOL), lambda i: (0, i))] * 3,
      out_specs=pl.BlockSpec((1, 1), lambda i: (0, 0),
                             memory_space=pltpu.SMEM),
      out_shape=jax.ShapeDtypeStruct((1, 1), jnp.float32),
      scratch_shapes=[pltpu.SMEM((2,), jnp.float32)],
      interpret=interpret,
  )


@functools.lru_cache(maxsize=None)
def _build():
  sc = _make_sc_kernel(_N, _G, _NC, _NS, _BLK)
  tc = _make_tc_kernel(_G, _NC, _G + 96)
  return sc, tc


def kernel(input, target, group_id):
  sc, tc = _build()
  cnt2, sum2, max2 = sc(input.astype(jnp.float32),
                        target.astype(jnp.float32),
                        group_id.astype(jnp.int32))
  ga = _G + 96
  out = tc(cnt2.reshape(_NC, ga), sum2.reshape(_NC, ga),
           max2.reshape(_NC, ga))
  return out[0, 0]


# speedup vs baseline: 1.4519x; 1.0155x over previous
import functools

import jax
import jax.numpy as jnp
from jax import lax
from jax.experimental import pallas as pl
from jax.experimental.pallas import tpu as pltpu
from jax.experimental.pallas import tpu_sc as plsc

_N = 6400000
_G = 100000
_NC = 2
_NS = 16
_BLK = 4000

_NEG = float("-inf")


def _make_sc_kernel(N, G, NC, NS, BLK, interpret=False):
  NW = NC * NS
  CHUNK = N // NW
  NB = CHUNK // BLK
  VPB = BLK // 16
  GA = G + 96
  WSL = GA // NS
  TRASH = G
  SENT = 2**31 - 1
  WOFF = 16
  LCAP = WOFF + BLK + 128
  NRMAX = (BLK + 127) // 128

  assert CHUNK % BLK == 0 and BLK % 128 == 0 or True
  assert N % NW == 0 and BLK % 16 == 0 and GA % NS == 0 and WSL % 8 == 0

  mesh = plsc.VectorSubcoreMesh(
      core_axis_name="c", subcore_axis_name="s",
      num_cores=NC, num_subcores=NS)

  def body(x_hbm, t_hbm, g_hbm, cnt_out, sum_out, max_out,
           gbuf, tbuf, xbuf, Lg, Lm, Lc, Lp,
           Lg2, Lmi2, Ssum2, Scnt2, Lmv2,
           gsc, msc, csc, psc, tmp16, zbuf, bgv, bmv,
           slots, rgbuf, rmbuf, recg, recm, ridx2, rval2,
           c_sh, s_sh, m_sh, bg_sh, bm_sh, dsem, fsem):
    core = lax.axis_index("c")
    sub = lax.axis_index("s")
    wid = core * NS + sub
    base = pl.multiple_of(wid * CHUNK, 8)
    iota = lax.broadcasted_iota(jnp.int32, (16,), 0)
    trashv = jnp.full((16,), TRASH, jnp.int32)

    def fill(ref, val, n, dtype):
      v = jnp.full((16,), val, dtype)
      def st(i, _):
        ref[pl.ds(i * 16, 16)] = v
        return 0
      lax.fori_loop(0, n // 16, st, 0)

    wsl0 = pl.multiple_of(sub * WSL, 8)
    fill(zbuf, 0.0, WSL, jnp.float32)
    pltpu.sync_copy(zbuf, c_sh.at[pl.ds(wsl0, WSL)])
    pltpu.sync_copy(zbuf, s_sh.at[pl.ds(wsl0, WSL)])
    fill(zbuf, _NEG, WSL, jnp.float32)
    pltpu.sync_copy(zbuf, m_sh.at[pl.ds(wsl0, WSL)])
    for u in range(5):
      gsc[pl.ds(u * 32, 16)] = jnp.full((16,), -1, jnp.int32)
      msc[pl.ds(u * 32, 16)] = jnp.zeros((16,), jnp.float32)
    plsc.subcore_barrier()

    pltpu.sync_copy(
        g_hbm.at[pl.ds(pl.multiple_of(jnp.maximum(base - 16, 0), 8), 16)],
        tmp16)
    prevg = tmp16[pl.ds(0, 16)][15]
    pltpu.sync_copy(g_hbm.at[pl.ds(base, 16)], tmp16)
    firstg = tmp16[pl.ds(0, 16)][0]
    fsg = jnp.where((prevg == firstg) & (wid > 0), firstg, jnp.int32(-1))
    fsgv = jnp.full((16,), fsg, jnp.int32)
    redv = jnp.full((16,), jnp.int32(G + 1) + sub, jnp.int32)

    def issue_loads(bb, pp):
      bel = pl.multiple_of(base + bb * BLK, 8)
      il = bel + BLK == N

      @pl.when(il)
      def _():
        pltpu.async_copy(g_hbm.at[pl.ds(bel, BLK)],
                         gbuf.at[pl.ds(pp * (BLK + 16), BLK)], dsem.at[pp])

      @pl.when(jnp.logical_not(il))
      def _():
        pltpu.async_copy(g_hbm.at[pl.ds(bel, BLK + 16)], gbuf.at[pl.ds(pp * (BLK + 16), BLK + 16)],
                         dsem.at[pp])

      pltpu.async_copy(t_hbm.at[pl.ds(bel, BLK)], tbuf.at[pl.ds(pp * BLK, BLK)], dsem.at[pp])
      pltpu.async_copy(x_hbm.at[pl.ds(bel, BLK)], xbuf.at[pl.ds(pp * BLK, BLK)], dsem.at[pp])

    def drain_loads(bb, pp):
      bel = pl.multiple_of(base + bb * BLK, 8)
      il = bel + BLK == N

      @pl.when(il)
      def _():
        pltpu.make_async_copy(g_hbm.at[pl.ds(bel, BLK)],
                              gbuf.at[pl.ds(pp * (BLK + 16), BLK)],
                              dsem.at[pp]).wait()

      @pl.when(jnp.logical_not(il))
      def _():
        pltpu.make_async_copy(g_hbm.at[pl.ds(bel, BLK + 16)], gbuf.at[pl.ds(pp * (BLK + 16), BLK + 16)],
                              dsem.at[pp]).wait()

      pltpu.make_async_copy(t_hbm.at[pl.ds(bel, BLK)], tbuf.at[pl.ds(pp * BLK, BLK)],
                            dsem.at[pp]).wait()
      pltpu.make_async_copy(x_hbm.at[pl.ds(bel, BLK)], xbuf.at[pl.ds(pp * BLK, BLK)],
                            dsem.at[pp]).wait()

    issue_loads(0, 0)

    def block_body(b, carry):
      pg, pm, cc, pcend, ppos = carry
      p = b & 1
      bel = pl.multiple_of(base + b * BLK, 8)
      is_last = bel + BLK == N

      @pl.when(b + 1 < NB)
      def _():
        issue_loads(b + 1, (b + 1) & 1)

      drain_loads(b, p)

      @pl.when(is_last)
      def _():
        gbuf[pl.ds(p * (BLK + 16) + BLK, 16)] = jnp.full((16,), SENT, jnp.int32)

      Lc[pl.ds(0, 16)] = jnp.full((16,), pcend, jnp.float32)
      Lp[pl.ds(0, 16)] = jnp.full((16,), ppos, jnp.int32)

      def vec_body(v, vc):
        vcc, off = vc
        for u in range(5):
          o = (v * 5 + u) * 16
          su = u * 32
          g = gbuf[pl.ds(p * (BLK + 16) + o, 16)]
          gn = gbuf[pl.ds(p * (BLK + 16) + o + 1, 16)]
          t = tbuf[pl.ds(p * BLK + o, 16)]
          x = xbuf[pl.ds(p * BLK + o, 16)]
          enc = lax.shift_left(g, 14) + (t * 16384.0).astype(jnp.int32)
          m = plsc.cummax(enc)
          c = plsc.cumsum(x) + jnp.full((16,), vcc, jnp.float32)
          e = (g != gn) | (iota == 15)
          offw = WOFF + off
          plsc.store_compressed(Lg.at[pl.ds(offw, 16)], g, mask=e)
          plsc.store_compressed(Lm.at[pl.ds(offw, 16)], m, mask=e)
          plsc.store_compressed(Lc.at[pl.ds(offw, 16)], c, mask=e)
          pos = iota + (b * BLK + o)
          plsc.store_compressed(Lp.at[pl.ds(offw, 16)], pos, mask=e)
          pc = plsc.all_reduce_population_count(e)
          off = off + pc[0]
          vcc = c[15]
        return (vcc, off)

      cc, off = lax.fori_loop(
          0, VPB // 5, vec_body, (cc, jnp.int32(0)))

      lkv = gbuf[pl.ds(p * (BLK + 16) + BLK - 1, 16)]
      cont = lkv[0] == lkv[1]
      supp = jnp.where(cont, off - 1, jnp.int32(-2))

      for k in range(8):
        Lg[pl.ds(WOFF + off + k * 16, 16)] = trashv

      nrows = lax.shift_right_logical(off + 127, 7)

      def row_body(j, rpm):
        for k in range(8):
          p0 = WOFF + j * 128 + k * 16
          lg = Lg[pl.ds(p0, 16)]
          lgn = Lg[pl.ds(p0 + 1, 16)]
          lc = Lc[pl.ds(p0, 16)]
          lcm = Lc[pl.ds(p0 - 1, 16)]
          lp = Lp[pl.ds(p0, 16)]
          lpm = Lp[pl.ds(p0 - 1, 16)]
          lm = Lm[pl.ds(p0, 16)]
          lm = jnp.maximum(lm, jnp.full((16,), rpm, jnp.int32))
          lm = plsc.cummax(lm)
          Lm[pl.ds(p0, 16)] = lm
          ridx = iota + (j * 128 + k * 16)
          is_end = (lg != lgn) & (ridx != supp)
          Lg2[j, pl.ds(k * 16, 16)] = lg
          Lmi2[j, pl.ds(k * 16, 16)] = jnp.where(
              is_end, jnp.where(lg == fsgv, redv, lg), trashv)
          Ssum2[j, pl.ds(k * 16, 16)] = lc - lcm
          Scnt2[j, pl.ds(k * 16, 16)] = (lp - lpm).astype(jnp.float32)
          Lmv2[j, pl.ds(k * 16, 16)] = (
              (lm & 16383).astype(jnp.float32) * (1.0 / 16384.0))
          rpm = lm[15]
        pltpu.async_copy(Ssum2.at[j], s_sh.at[Lg2.at[j]], fsem, add=True)
        pltpu.async_copy(Scnt2.at[j], c_sh.at[Lg2.at[j]], fsem, add=True)
        pltpu.async_copy(Lmv2.at[j], m_sh.at[Lmi2.at[j]], fsem)
        return rpm

      lax.fori_loop(0, nrows, row_body, pm)

      def row_wait(j, _):
        pltpu.make_async_copy(Ssum2.at[j], s_sh.at[Lg2.at[j]], fsem).wait()
        pltpu.make_async_copy(Scnt2.at[j], c_sh.at[Lg2.at[j]], fsem).wait()
        pltpu.make_async_copy(Lmv2.at[j], m_sh.at[Lmi2.at[j]], fsem).wait()
        return 0

      lax.fori_loop(0, nrows, row_wait, 0)

      pcend2 = Lc[pl.ds(off, 16)][15]
      ppos2 = Lp[pl.ds(off, 16)][15]
      pg2 = Lg[pl.ds(off, 16)][15]
      pm2 = Lm[pl.ds(off, 16)][15]
      return (pg2, pm2, cc, pcend2, ppos2)

    cg, cme, cc, pcend, ppos = lax.fori_loop(
        0, NB, block_body,
        (jnp.int32(-1), jnp.int32(-1), jnp.float32(0.0),
         jnp.float32(0.0), jnp.int32(-1)))
    cm = (cme & 16383).astype(jnp.float32) * (1.0 / 16384.0)

    del cc, pcend, ppos

    fsg_rec = jnp.where(fsg == -1, TRASH, fsg)
    bg = jnp.where(iota == 0, jnp.full((16,), fsg_rec, jnp.int32),
                   jnp.where(iota == 1, jnp.full((16,), cg, jnp.int32),
                             trashv))
    bm = jnp.where(iota == 1, jnp.full((16,), cm, jnp.float32),
                   jnp.full((16,), _NEG, jnp.float32))
    bgv[pl.ds(0, 16)] = bg
    bmv[pl.ds(0, 16)] = bm
    b8 = pl.multiple_of(sub * 8, 8)
    pltpu.sync_copy(bgv.at[pl.ds(0, 8)], bg_sh.at[pl.ds(b8, 8)])
    pltpu.sync_copy(bmv.at[pl.ds(0, 8)], bm_sh.at[pl.ds(b8, 8)])
    plsc.subcore_barrier()

    @pl.when(sub == 0)
    def _():
      pltpu.sync_copy(bg_sh, rgbuf)
      pltpu.sync_copy(bm_sh, rmbuf)
      pltpu.sync_copy(m_sh.at[pl.ds(G, 32)], slots)
      recg[pl.ds(32, 16)] = jnp.full((16,), SENT, jnp.int32)
      half = lax.shift_right_logical(iota, 1)
      odd = (iota & 1) == 1
      for r in range(2):
        sl = half + r * 8
        src = sl * 8 + (iota & 1)
        gvec = plsc.load_gather(rgbuf, [src])
        recg[pl.ds(16 * r, 16)] = gvec
        m_even = plsc.load_gather(slots, [sl + 1])
        m_odd = plsc.load_gather(rmbuf, [sl * 8 + 1])
        recm[pl.ds(16 * r, 16)] = jnp.where(odd, m_odd, m_even)
      ccg = jnp.int32(-1)
      ccm = _NEG
      for r in range(2):
        g = recg[pl.ds(16 * r, 16)]
        m0 = recm[pl.ds(16 * r, 16)]
        gn = recg[pl.ds(16 * r + 1, 16)]
        m = jnp.where(g == jnp.full((16,), ccg, jnp.int32),
                      jnp.maximum(m0, jnp.full((16,), ccm, jnp.float32)), m0)
        gsc[pl.ds(16, 16)] = g
        for d in (1, 2, 4, 8):
          msc[pl.ds(16, 16)] = m
          gs = gsc[pl.ds(16 - d, 16)]
          ms = msc[pl.ds(16 - d, 16)]
          m = jnp.where(g == gs, jnp.maximum(m, ms), m)
        e = g != gn
        ridx2[r, pl.ds(0, 16)] = jnp.where(e, g, trashv)
        rval2[r, pl.ds(0, 16)] = m
        ccg = g[15]
        ccm = m[15]
      for r in range(2):
        pltpu.sync_copy(rval2.at[r], m_sh.at[ridx2.at[r]])

    plsc.subcore_barrier()

    ob = pl.multiple_of(core * GA + wsl0, 8)
    for sh, out in ((c_sh, cnt_out), (s_sh, sum_out), (m_sh, max_out)):
      pltpu.sync_copy(sh.at[pl.ds(wsl0, WSL)], zbuf)
      pltpu.sync_copy(zbuf, out.at[pl.ds(ob, WSL)])

  f32 = jnp.float32
  i32 = jnp.int32
  out_type = [jax.ShapeDtypeStruct((NC * GA,), f32)] * 3
  scratch = [
      pltpu.VMEM((2 * (BLK + 16),), i32),
      pltpu.VMEM((2 * BLK,), f32),
      pltpu.VMEM((2 * BLK,), f32),
      pltpu.VMEM((LCAP,), i32),
      pltpu.VMEM((LCAP,), i32),
      pltpu.VMEM((LCAP,), f32),
      pltpu.VMEM((LCAP,), i32),
      pltpu.VMEM((NRMAX, 128), i32),
      pltpu.VMEM((NRMAX, 128), i32),
      pltpu.VMEM((NRMAX, 128), f32),
      pltpu.VMEM((NRMAX, 128), f32),
      pltpu.VMEM((NRMAX, 128), f32),
      pltpu.VMEM((160,), i32),
      pltpu.VMEM((160,), f32),
      pltpu.VMEM((16,), f32),
      pltpu.VMEM((16,), i32),
      pltpu.VMEM((16,), i32),
      pltpu.VMEM((WSL,), f32),
      pltpu.VMEM((16,), i32),
      pltpu.VMEM((16,), f32),
      pltpu.VMEM((32,), f32),
      pltpu.VMEM((128,), i32),
      pltpu.VMEM((128,), f32),
      pltpu.VMEM((48,), i32),
      pltpu.VMEM((48,), f32),
      pltpu.VMEM((2, 16), i32),
      pltpu.VMEM((2, 16), f32),
      pltpu.VMEM_SHARED((GA,), f32),
      pltpu.VMEM_SHARED((GA,), f32),
      pltpu.VMEM_SHARED((GA,), f32),
      pltpu.VMEM_SHARED((128,), i32),
      pltpu.VMEM_SHARED((128,), f32),
      pltpu.SemaphoreType.DMA((2,)),
      pltpu.SemaphoreType.DMA,
  ]
  return pl.kernel(body, out_type=out_type, mesh=mesh,
                   scratch_types=scratch,
                   compiler_params=pltpu.CompilerParams(
                       needs_layout_passes=False),
                   interpret=interpret)


def _make_tc_kernel(G, NC, GA, interpret=False):
  BCOL = 5888
  NSTEP = GA // BCOL
  assert NSTEP * BCOL == GA

  def tc_body(c_ref, s_ref, m_ref, out_ref, acc_ref):
    i = pl.program_id(0)

    @pl.when(i == 0)
    def _():
      acc_ref[0] = 0.0
      acc_ref[1] = 0.0

    c = c_ref[0:1, :] + c_ref[1:2, :]
    s = s_ref[0:1, :] + s_ref[1:2, :]
    m = jnp.maximum(m_ref[0:1, :], m_ref[1:2, :])
    col = lax.broadcasted_iota(jnp.int32, (1, BCOL), 1) + i * BCOL
    valid = (col < G) & (c > 0.0)
    mean = jnp.where(valid, s / jnp.maximum(c, 1.0), 0.0)
    tm = jnp.where(valid, m, 0.0)

    def lgs(x):
      return jnp.minimum(x, 0.0) - jnp.log1p(jnp.exp(-jnp.abs(x)))

    per = tm * lgs(mean) + (1.0 - tm) * lgs(1.0 - mean)
    per = jnp.where(valid, per, 0.0)
    acc_ref[0] += jnp.sum(per)
    acc_ref[1] += jnp.sum(valid.astype(jnp.float32))

    @pl.when(i == NSTEP - 1)
    def _():
      out_ref[0, 0] = -acc_ref[0] / acc_ref[1]

  return pl.pallas_call(
      tc_body,
      grid=(NSTEP,),
      in_specs=[pl.BlockSpec((NC, BCOL), lambda i: (0, i))] * 3,
      out_specs=pl.BlockSpec((1, 1), lambda i: (0, 0),
                             memory_space=pltpu.SMEM),
      out_shape=jax.ShapeDtypeStruct((1, 1), jnp.float32),
      scratch_shapes=[pltpu.SMEM((2,), jnp.float32)],
      interpret=interpret,
  )


@functools.lru_cache(maxsize=None)
def _build():
  sc = _make_sc_kernel(_N, _G, _NC, _NS, _BLK)
  tc = _make_tc_kernel(_G, _NC, _G + 96)
  return sc, tc


def kernel(input, target, group_id):
  sc, tc = _build()
  cnt2, sum2, max2 = sc(input.astype(jnp.float32),
                        target.astype(jnp.float32),
                        group_id.astype(jnp.int32))
  ga = _G + 96
  out = tc(cnt2.reshape(_NC, ga), sum2.reshape(_NC, ga),
           max2.reshape(_NC, ga))
  return out[0, 0]


# speedup vs baseline: 1.4542x; 1.0016x over previous
import functools

import jax
import jax.numpy as jnp
from jax import lax
from jax.experimental import pallas as pl
from jax.experimental.pallas import tpu as pltpu
from jax.experimental.pallas import tpu_sc as plsc

_N = 6400000
_G = 100000
_NC = 2
_NS = 16
_BLK = 4000

_NEG = float("-inf")


def _make_sc_kernel(N, G, NC, NS, BLK, interpret=False):
  NW = NC * NS
  CHUNK = N // NW
  NB = CHUNK // BLK
  VPB = BLK // 16
  GA = G + 96
  WSL = GA // NS
  TRASH = G
  SENT = 2**31 - 1
  WOFF = 16
  LCAP = WOFF + BLK + 128
  NRMAX = (BLK + 127) // 128

  assert CHUNK % BLK == 0 and BLK % 128 == 0 or True
  assert N % NW == 0 and BLK % 16 == 0 and GA % NS == 0 and WSL % 8 == 0

  mesh = plsc.VectorSubcoreMesh(
      core_axis_name="c", subcore_axis_name="s",
      num_cores=NC, num_subcores=NS)

  def body(x_hbm, t_hbm, g_hbm, cnt_out, sum_out, max_out,
           gbuf, tbuf, xbuf, Lg, Lm, Lc, Lp,
           Lg2, Lmi2, Ssum2, Scnt2, Lmv2,
           gsc, msc, csc, psc, tmp16, zbuf, bgv, bmv,
           slots, rgbuf, rmbuf, recg, recm, ridx2, rval2,
           c_sh, s_sh, m_sh, bg_sh, bm_sh, dsem, fsem):
    core = lax.axis_index("c")
    sub = lax.axis_index("s")
    wid = core * NS + sub
    base = pl.multiple_of(wid * CHUNK, 8)
    iota = lax.broadcasted_iota(jnp.int32, (16,), 0)
    trashv = jnp.full((16,), TRASH, jnp.int32)

    def fill(ref, val, n, dtype):
      v = jnp.full((16,), val, dtype)
      def st(i, _):
        ref[pl.ds(i * 16, 16)] = v
        return 0
      lax.fori_loop(0, n // 16, st, 0)

    wsl0 = pl.multiple_of(sub * WSL, 8)
    fill(zbuf, 0.0, WSL, jnp.float32)
    pltpu.sync_copy(zbuf, c_sh.at[pl.ds(wsl0, WSL)])
    pltpu.sync_copy(zbuf, s_sh.at[pl.ds(wsl0, WSL)])
    fill(zbuf, _NEG, WSL, jnp.float32)
    pltpu.sync_copy(zbuf, m_sh.at[pl.ds(wsl0, WSL)])
    for u in range(5):
      gsc[pl.ds(u * 32, 16)] = jnp.full((16,), -1, jnp.int32)
      msc[pl.ds(u * 32, 16)] = jnp.zeros((16,), jnp.float32)
    plsc.subcore_barrier()

    pltpu.sync_copy(
        g_hbm.at[pl.ds(pl.multiple_of(jnp.maximum(base - 16, 0), 8), 16)],
        tmp16)
    prevg = tmp16[pl.ds(0, 16)][15]
    pltpu.sync_copy(g_hbm.at[pl.ds(base, 16)], tmp16)
    firstg = tmp16[pl.ds(0, 16)][0]
    fsg = jnp.where((prevg == firstg) & (wid > 0), firstg, jnp.int32(-1))
    fsgv = jnp.full((16,), fsg, jnp.int32)
    redv = jnp.full((16,), jnp.int32(G + 1) + sub, jnp.int32)

    def issue_loads(bb, pp):
      bel = pl.multiple_of(base + bb * BLK, 8)
      il = bel + BLK == N

      @pl.when(il)
      def _():
        pltpu.async_copy(g_hbm.at[pl.ds(bel, BLK)],
                         gbuf.at[pl.ds(pp * (BLK + 16), BLK)], dsem.at[pp])

      @pl.when(jnp.logical_not(il))
      def _():
        pltpu.async_copy(g_hbm.at[pl.ds(bel, BLK + 16)], gbuf.at[pl.ds(pp * (BLK + 16), BLK + 16)],
                         dsem.at[pp])

      pltpu.async_copy(t_hbm.at[pl.ds(bel, BLK)], tbuf.at[pl.ds(pp * BLK, BLK)], dsem.at[pp])
      pltpu.async_copy(x_hbm.at[pl.ds(bel, BLK)], xbuf.at[pl.ds(pp * BLK, BLK)], dsem.at[pp])

    def drain_loads(bb, pp):
      bel = pl.multiple_of(base + bb * BLK, 8)
      il = bel + BLK == N

      @pl.when(il)
      def _():
        pltpu.make_async_copy(g_hbm.at[pl.ds(bel, BLK)],
                              gbuf.at[pl.ds(pp * (BLK + 16), BLK)],
                              dsem.at[pp]).wait()

      @pl.when(jnp.logical_not(il))
      def _():
        pltpu.make_async_copy(g_hbm.at[pl.ds(bel, BLK + 16)], gbuf.at[pl.ds(pp * (BLK + 16), BLK + 16)],
                              dsem.at[pp]).wait()

      pltpu.make_async_copy(t_hbm.at[pl.ds(bel, BLK)], tbuf.at[pl.ds(pp * BLK, BLK)],
                            dsem.at[pp]).wait()
      pltpu.make_async_copy(x_hbm.at[pl.ds(bel, BLK)], xbuf.at[pl.ds(pp * BLK, BLK)],
                            dsem.at[pp]).wait()

    issue_loads(0, 0)

    def block_body(b, carry):
      pg, pm, ppos = carry
      p = b & 1
      bel = pl.multiple_of(base + b * BLK, 8)
      is_last = bel + BLK == N

      @pl.when(b + 1 < NB)
      def _():
        issue_loads(b + 1, (b + 1) & 1)

      drain_loads(b, p)

      @pl.when(is_last)
      def _():
        gbuf[pl.ds(p * (BLK + 16) + BLK, 16)] = jnp.full((16,), SENT, jnp.int32)

      Lp[pl.ds(0, 16)] = jnp.full((16,), ppos, jnp.int32)

      def vec_body(v, off):
        for u in range(5):
          o = (v * 5 + u) * 16
          su = u * 32
          g = gbuf[pl.ds(p * (BLK + 16) + o, 16)]
          gn = gbuf[pl.ds(p * (BLK + 16) + o + 1, 16)]
          t = tbuf[pl.ds(p * BLK + o, 16)]
          x = xbuf[pl.ds(p * BLK + o, 16)]
          enc = lax.shift_left(g, 14) + (t * 16384.0).astype(jnp.int32)
          m = plsc.cummax(enc)
          c = plsc.cumsum(x)
          e = (g != gn) | (iota == 15)
          offw = WOFF + off
          plsc.store_compressed(Lg.at[pl.ds(offw, 16)], g, mask=e)
          plsc.store_compressed(Lm.at[pl.ds(offw, 16)], m, mask=e)
          plsc.store_compressed(Lc.at[pl.ds(offw, 16)], c, mask=e)
          pos = iota + (b * BLK + o)
          plsc.store_compressed(Lp.at[pl.ds(offw, 16)], pos, mask=e)
          pc = plsc.all_reduce_population_count(e)
          off = off + pc[0]
        return off

      off = lax.fori_loop(0, VPB // 5, vec_body, jnp.int32(0))

      lkv = gbuf[pl.ds(p * (BLK + 16) + BLK - 1, 16)]
      cont = lkv[0] == lkv[1]
      supp = jnp.where(cont, off - 1, jnp.int32(-2))

      for k in range(8):
        Lg[pl.ds(WOFF + off + k * 16, 16)] = trashv

      nrows = lax.shift_right_logical(off + 127, 7)

      def row_body(j, rpm):
        for k in range(8):
          p0 = WOFF + j * 128 + k * 16
          lg = Lg[pl.ds(p0, 16)]
          lgn = Lg[pl.ds(p0 + 1, 16)]
          lc = Lc[pl.ds(p0, 16)]
          lcm = Lc[pl.ds(p0 - 1, 16)]
          lp = Lp[pl.ds(p0, 16)]
          lpm = Lp[pl.ds(p0 - 1, 16)]
          csub = jnp.where(
              lax.shift_right_arithmetic(lp, 4)
              == lax.shift_right_arithmetic(lpm, 4), lcm, 0.0)
          lm = Lm[pl.ds(p0, 16)]
          lm = jnp.maximum(lm, jnp.full((16,), rpm, jnp.int32))
          lm = plsc.cummax(lm)
          Lm[pl.ds(p0, 16)] = lm
          ridx = iota + (j * 128 + k * 16)
          is_end = (lg != lgn) & (ridx != supp)
          Lg2[j, pl.ds(k * 16, 16)] = lg
          Lmi2[j, pl.ds(k * 16, 16)] = jnp.where(
              is_end, jnp.where(lg == fsgv, redv, lg), trashv)
          Ssum2[j, pl.ds(k * 16, 16)] = lc - csub
          Scnt2[j, pl.ds(k * 16, 16)] = (lp - lpm).astype(jnp.float32)
          Lmv2[j, pl.ds(k * 16, 16)] = (
              (lm & 16383).astype(jnp.float32) * (1.0 / 16384.0))
          rpm = lm[15]
        pltpu.async_copy(Ssum2.at[j], s_sh.at[Lg2.at[j]], fsem, add=True)
        pltpu.async_copy(Scnt2.at[j], c_sh.at[Lg2.at[j]], fsem, add=True)
        pltpu.async_copy(Lmv2.at[j], m_sh.at[Lmi2.at[j]], fsem)
        return rpm

      lax.fori_loop(0, nrows, row_body, pm)

      def row_wait(j, _):
        pltpu.make_async_copy(Ssum2.at[j], s_sh.at[Lg2.at[j]], fsem).wait()
        pltpu.make_async_copy(Scnt2.at[j], c_sh.at[Lg2.at[j]], fsem).wait()
        pltpu.make_async_copy(Lmv2.at[j], m_sh.at[Lmi2.at[j]], fsem).wait()
        return 0

      lax.fori_loop(0, nrows, row_wait, 0)

      ppos2 = Lp[pl.ds(off, 16)][15]
      pg2 = Lg[pl.ds(off, 16)][15]
      pm2 = Lm[pl.ds(off, 16)][15]
      return (pg2, pm2, ppos2)

    cg, cme, ppos = lax.fori_loop(
        0, NB, block_body,
        (jnp.int32(-1), jnp.int32(-1), jnp.int32(-1)))
    cm = (cme & 16383).astype(jnp.float32) * (1.0 / 16384.0)
    del ppos

    fsg_rec = jnp.where(fsg == -1, TRASH, fsg)
    bg = jnp.where(iota == 0, jnp.full((16,), fsg_rec, jnp.int32),
                   jnp.where(iota == 1, jnp.full((16,), cg, jnp.int32),
                             trashv))
    bm = jnp.where(iota == 1, jnp.full((16,), cm, jnp.float32),
                   jnp.full((16,), _NEG, jnp.float32))
    bgv[pl.ds(0, 16)] = bg
    bmv[pl.ds(0, 16)] = bm
    b8 = pl.multiple_of(sub * 8, 8)
    pltpu.sync_copy(bgv.at[pl.ds(0, 8)], bg_sh.at[pl.ds(b8, 8)])
    pltpu.sync_copy(bmv.at[pl.ds(0, 8)], bm_sh.at[pl.ds(b8, 8)])
    plsc.subcore_barrier()

    @pl.when(sub == 0)
    def _():
      pltpu.sync_copy(bg_sh, rgbuf)
      pltpu.sync_copy(bm_sh, rmbuf)
      pltpu.sync_copy(m_sh.at[pl.ds(G, 32)], slots)
      recg[pl.ds(32, 16)] = jnp.full((16,), SENT, jnp.int32)
      half = lax.shift_right_logical(iota, 1)
      odd = (iota & 1) == 1
      for r in range(2):
        sl = half + r * 8
        src = sl * 8 + (iota & 1)
        gvec = plsc.load_gather(rgbuf, [src])
        recg[pl.ds(16 * r, 16)] = gvec
        m_even = plsc.load_gather(slots, [sl + 1])
        m_odd = plsc.load_gather(rmbuf, [sl * 8 + 1])
        recm[pl.ds(16 * r, 16)] = jnp.where(odd, m_odd, m_even)
      ccg = jnp.int32(-1)
      ccm = _NEG
      for r in range(2):
        g = recg[pl.ds(16 * r, 16)]
        m0 = recm[pl.ds(16 * r, 16)]
        gn = recg[pl.ds(16 * r + 1, 16)]
        m = jnp.where(g == jnp.full((16,), ccg, jnp.int32),
                      jnp.maximum(m0, jnp.full((16,), ccm, jnp.float32)), m0)
        gsc[pl.ds(16, 16)] = g
        for d in (1, 2, 4, 8):
          msc[pl.ds(16, 16)] = m
          gs = gsc[pl.ds(16 - d, 16)]
          ms = msc[pl.ds(16 - d, 16)]
          m = jnp.where(g == gs, jnp.maximum(m, ms), m)
        e = g != gn
        ridx2[r, pl.ds(0, 16)] = jnp.where(e, g, trashv)
        rval2[r, pl.ds(0, 16)] = m
        ccg = g[15]
        ccm = m[15]
      for r in range(2):
        pltpu.sync_copy(rval2.at[r], m_sh.at[ridx2.at[r]])

    plsc.subcore_barrier()

    ob = pl.multiple_of(core * GA + wsl0, 8)
    for sh, out in ((c_sh, cnt_out), (s_sh, sum_out), (m_sh, max_out)):
      pltpu.sync_copy(sh.at[pl.ds(wsl0, WSL)], zbuf)
      pltpu.sync_copy(zbuf, out.at[pl.ds(ob, WSL)])

  f32 = jnp.float32
  i32 = jnp.int32
  out_type = [jax.ShapeDtypeStruct((NC * GA,), f32)] * 3
  scratch = [
      pltpu.VMEM((2 * (BLK + 16),), i32),
      pltpu.VMEM((2 * BLK,), f32),
      pltpu.VMEM((2 * BLK,), f32),
      pltpu.VMEM((LCAP,), i32),
      pltpu.VMEM((LCAP,), i32),
      pltpu.VMEM((LCAP,), f32),
      pltpu.VMEM((LCAP,), i32),
      pltpu.VMEM((NRMAX, 128), i32),
      pltpu.VMEM((NRMAX, 128), i32),
      pltpu.VMEM((NRMAX, 128), f32),
      pltpu.VMEM((NRMAX, 128), f32),
      pltpu.VMEM((NRMAX, 128), f32),
      pltpu.VMEM((160,), i32),
      pltpu.VMEM((160,), f32),
      pltpu.VMEM((16,), f32),
      pltpu.VMEM((16,), i32),
      pltpu.VMEM((16,), i32),
      pltpu.VMEM((WSL,), f32),
      pltpu.VMEM((16,), i32),
      pltpu.VMEM((16,), f32),
      pltpu.VMEM((32,), f32),
      pltpu.VMEM((128,), i32),
      pltpu.VMEM((128,), f32),
      pltpu.VMEM((48,), i32),
      pltpu.VMEM((48,), f32),
      pltpu.VMEM((2, 16), i32),
      pltpu.VMEM((2, 16), f32),
      pltpu.VMEM_SHARED((GA,), f32),
      pltpu.VMEM_SHARED((GA,), f32),
      pltpu.VMEM_SHARED((GA,), f32),
      pltpu.VMEM_SHARED((128,), i32),
      pltpu.VMEM_SHARED((128,), f32),
      pltpu.SemaphoreType.DMA((2,)),
      pltpu.SemaphoreType.DMA,
  ]
  return pl.kernel(body, out_type=out_type, mesh=mesh,
                   scratch_types=scratch,
                   compiler_params=pltpu.CompilerParams(
                       needs_layout_passes=False),
                   interpret=interpret)


def _make_tc_kernel(G, NC, GA, interpret=False):
  BCOL = 5888
  NSTEP = GA // BCOL
  assert NSTEP * BCOL == GA

  def tc_body(c_ref, s_ref, m_ref, out_ref, acc_ref):
    i = pl.program_id(0)

    @pl.when(i == 0)
    def _():
      acc_ref[0] = 0.0
      acc_ref[1] = 0.0

    c = c_ref[0:1, :] + c_ref[1:2, :]
    s = s_ref[0:1, :] + s_ref[1:2, :]
    m = jnp.maximum(m_ref[0:1, :], m_ref[1:2, :])
    col = lax.broadcasted_iota(jnp.int32, (1, BCOL), 1) + i * BCOL
    valid = (col < G) & (c > 0.0)
    mean = jnp.where(valid, s / jnp.maximum(c, 1.0), 0.0)
    tm = jnp.where(valid, m, 0.0)

    def lgs(x):
      return jnp.minimum(x, 0.0) - jnp.log1p(jnp.exp(-jnp.abs(x)))

    per = tm * lgs(mean) + (1.0 - tm) * lgs(1.0 - mean)
    per = jnp.where(valid, per, 0.0)
    acc_ref[0] += jnp.sum(per)
    acc_ref[1] += jnp.sum(valid.astype(jnp.float32))

    @pl.when(i == NSTEP - 1)
    def _():
      out_ref[0, 0] = -acc_ref[0] / acc_ref[1]

  return pl.pallas_call(
      tc_body,
      grid=(NSTEP,),
      in_specs=[pl.BlockSpec((NC, BCOL), lambda i: (0, i))] * 3,
      out_specs=pl.BlockSpec((1, 1), lambda i: (0, 0),
                             memory_space=pltpu.SMEM),
      out_shape=jax.ShapeDtypeStruct((1, 1), jnp.float32),
      scratch_shapes=[pltpu.SMEM((2,), jnp.float32)],
      interpret=interpret,
  )


@functools.lru_cache(maxsize=None)
def _build():
  sc = _make_sc_kernel(_N, _G, _NC, _NS, _BLK)
  tc = _make_tc_kernel(_G, _NC, _G + 96)
  return sc, tc


def kernel(input, target, group_id):
  sc, tc = _build()
  cnt2, sum2, max2 = sc(input.astype(jnp.float32),
                        target.astype(jnp.float32),
                        group_id.astype(jnp.int32))
  ga = _G + 96
  out = tc(cnt2.reshape(_NC, ga), sum2.reshape(_NC, ga),
           max2.reshape(_NC, ga))
  return out[0, 0]
